# baseline (speedup 1.0000x reference)
import jax
import jax.numpy as jnp
from jax.experimental import pallas as pl

PRE_NMS_THRESH = 0.05
PRE_NMS_TOP_N = 1000


def _score_body(cls_ref, out_ref):
    x = cls_ref[...]
    s = jax.nn.sigmoid(x)
    out_ref[...] = jnp.where(s > PRE_NMS_THRESH, s, 0.0)


def kernel(locations, box_cls, box_regression, centerness, kps_pred,
           heatmaps_coords, heatmaps, image_sizes, stride):
    N, C, H, W = box_cls.shape
    NK = kps_pred.shape[1] // 2
    HW = H * W

    cls2d = box_cls.reshape(C, HW)
    masked = pl.pallas_call(
        _score_body,
        out_shape=jax.ShapeDtypeStruct((C, HW), jnp.float32),
    )(cls2d)

    flat = masked.T.reshape(-1)
    top_scores, top_idx = jax.lax.top_k(flat, PRE_NMS_TOP_N)
    loc_idx = top_idx // C

    kps = kps_pred.reshape(2 * NK, HW).T.reshape(HW, NK, 2)
    k_sel = kps[loc_idx]
    l_sel = locations[loc_idx]
    kabs = k_sel + l_sel[:, None, :]
    v = jnp.ones((PRE_NMS_TOP_N, NK, 1), dtype=kabs.dtype)
    kps3 = jnp.concatenate([kabs, v], axis=-1)
    lt = jnp.min(kabs, axis=1)
    rb = jnp.max(kabs, axis=1)
    h = image_sizes[0, 0].astype(jnp.float32)
    w = image_sizes[0, 1].astype(jnp.float32)
    x1 = jnp.clip(lt[:, 0], 0.0, w - 1.0)
    y1 = jnp.clip(lt[:, 1], 0.0, h - 1.0)
    x2 = jnp.clip(rb[:, 0], 0.0, w - 1.0)
    y2 = jnp.clip(rb[:, 1], 0.0, h - 1.0)
    det = jnp.stack([x1, y1, x2, y2], axis=1)
    scores = top_scores
    out = jnp.concatenate([det, scores[:, None],
                           kps3.reshape(PRE_NMS_TOP_N, -1)], axis=1)
    return out


# speedup vs baseline: 3.3766x; 3.3766x over previous
import jax
import jax.numpy as jnp
from jax.experimental import pallas as pl
from jax.experimental.pallas import tpu as pltpu

PRE_NMS_THRESH = 0.05
PRE_NMS_TOP_N = 1000
CAND_BUF = 2048
ROWS = 1280
COLS = 1024
BISECT_ITERS = 31
KEY_HI = 0x3F800002


def _score_thresh_body(x_ref, scores_ref, thresh_ref, cnt_ref):
    x = x_ref[...]
    s = jax.nn.sigmoid(x)
    masked = jnp.where(s > PRE_NMS_THRESH, s, 0.0)
    scores_ref[...] = masked

    def body(_, carry):
        lo, hi = carry
        mid = (lo + hi) // 2
        t = jax.lax.bitcast_convert_type(mid, jnp.float32)
        cnt = jnp.sum((masked >= t).astype(jnp.float32))
        pred = cnt >= float(PRE_NMS_TOP_N)
        return (jnp.where(pred, mid, lo), jnp.where(pred, hi, mid))

    lo, _ = jax.lax.fori_loop(
        0, BISECT_ITERS, body,
        (jnp.int32(0), jnp.int32(KEY_HI)))
    t_final = jax.lax.bitcast_convert_type(lo, jnp.float32)
    thresh_ref[0, 0] = t_final
    cnt_ref[0, 0] = jnp.sum((masked >= t_final).astype(jnp.float32))


def kernel(locations, box_cls, box_regression, centerness, kps_pred,
           heatmaps_coords, heatmaps, image_sizes, stride):
    N, C, H, W = box_cls.shape
    NK = kps_pred.shape[1] // 2
    HW = H * W

    cls_t = box_cls.reshape(C, HW).T.reshape(ROWS, COLS)

    scores, thresh, _cnt = pl.pallas_call(
        _score_thresh_body,
        out_shape=[
            jax.ShapeDtypeStruct((ROWS, COLS), jnp.float32),
            jax.ShapeDtypeStruct((1, 1), jnp.float32),
            jax.ShapeDtypeStruct((1, 1), jnp.float32),
        ],
        out_specs=[
            pl.BlockSpec((ROWS, COLS), lambda: (0, 0)),
            pl.BlockSpec(memory_space=pltpu.SMEM),
            pl.BlockSpec(memory_space=pltpu.SMEM),
        ],
        in_specs=[pl.BlockSpec((ROWS, COLS), lambda: (0, 0))],
    )(cls_t)

    flat = scores.reshape(-1)
    t = thresh[0, 0]
    mask = flat >= t
    cand_idx = jnp.nonzero(mask, size=CAND_BUF, fill_value=0)[0]
    n_cand = jnp.sum(mask.astype(jnp.int32))
    valid = jnp.arange(CAND_BUF, dtype=jnp.int32) < n_cand
    cand_s = jnp.where(valid, flat[cand_idx], 0.0)

    top_scores, pos = jax.lax.top_k(cand_s, PRE_NMS_TOP_N)
    top_idx = cand_idx[pos]
    loc_idx = top_idx // C

    kps = kps_pred.reshape(2 * NK, HW).T.reshape(HW, NK, 2)
    k_sel = kps[loc_idx]
    l_sel = locations[loc_idx]
    kabs = k_sel + l_sel[:, None, :]
    v = jnp.ones((PRE_NMS_TOP_N, NK, 1), dtype=kabs.dtype)
    kps3 = jnp.concatenate([kabs, v], axis=-1)
    lt = jnp.min(kabs, axis=1)
    rb = jnp.max(kabs, axis=1)
    h = image_sizes[0, 0].astype(jnp.float32)
    w = image_sizes[0, 1].astype(jnp.float32)
    x1 = jnp.clip(lt[:, 0], 0.0, w - 1.0)
    y1 = jnp.clip(lt[:, 1], 0.0, h - 1.0)
    x2 = jnp.clip(rb[:, 0], 0.0, w - 1.0)
    y2 = jnp.clip(rb[:, 1], 0.0, h - 1.0)
    det = jnp.stack([x1, y1, x2, y2], axis=1)
    out = jnp.concatenate([det, top_scores[:, None],
                           kps3.reshape(PRE_NMS_TOP_N, -1)], axis=1)
    return out
